# baseline (speedup 1.0000x reference)
import functools

import jax
import jax.numpy as jnp
from jax.experimental import pallas as pl
from jax.experimental.pallas import tpu as pltpu

_LANES = 128
_N_SUB = 4


def _bnneck_kernel(x_hbm, w_hbm, gamma_ref, beta_ref, o_hbm,
                   x_st, x_asm, w_st, z_buf, sem_x, sem_w, sem_out,
                   *, n, c_in, c_out):
    co_half = c_out // 2
    sub = co_half // _N_SUB
    i = pl.program_id(0)
    co_base = i * co_half

    def w_load(s):
        return pltpu.make_async_copy(
            w_hbm.at[pl.ds(co_base + s * sub, sub)], w_st.at[s], sem_w.at[s])

    def out_store(s):
        return pltpu.make_async_copy(
            z_buf.at[s % 2], o_hbm.at[:, pl.ds(co_base + s * sub, sub)],
            sem_out.at[s % 2])

    x_load = pltpu.make_async_copy(x_hbm, x_st, sem_x)
    x_load.start()
    for s in range(_N_SUB):
        w_load(s).start()

    x_load.wait()
    x_asm[...] = x_st[...].astype(jnp.bfloat16).reshape(n, c_in)

    inv_n = 1.0 / float(n)
    for s in range(_N_SUB):
        w_load(s).wait()
        wk = w_st[s].astype(jnp.bfloat16).reshape(sub, c_in)
        y = jax.lax.dot_general(
            x_asm[...], wk, dimension_numbers=(((1,), (1,)), ((), ())),
            preferred_element_type=jnp.float32)
        mean = jnp.sum(y, axis=0, keepdims=True) * inv_n
        diff = y - mean
        var = jnp.sum(diff * diff, axis=0, keepdims=True) * inv_n
        z = diff * jax.lax.rsqrt(var + 1e-5)
        cs = pl.ds(s * sub, sub)
        z = z * gamma_ref[:, cs] + beta_ref[:, cs]
        if s >= 2:
            out_store(s - 2).wait()
        z_buf[s % 2, :, :] = jnp.where(z >= 0, z, 0.25 * z)
        out_store(s).start()
    for s in range(max(0, _N_SUB - 2), _N_SUB):
        out_store(s).wait()


def kernel(x, weight, gamma, beta):
    n, c_in, h, w_sp = x.shape
    assert h == 1 and w_sp == 1
    c_out = weight.shape[0]
    assert n % 8 == 0 and c_in % _LANES == 0
    assert c_out % (2 * _N_SUB * _LANES) == 0
    kj = c_in // _LANES
    co_half = c_out // 2
    sub = co_half // _N_SUB

    x3 = x.reshape(n, kj, _LANES)
    w3 = weight.reshape(c_out, kj, _LANES)
    gamma2 = gamma.reshape(1, c_out).astype(jnp.float32)
    beta2 = beta.reshape(1, c_out).astype(jnp.float32)

    body = functools.partial(_bnneck_kernel, n=n, c_in=c_in, c_out=c_out)
    return pl.pallas_call(
        body,
        out_shape=jax.ShapeDtypeStruct((n, c_out), x.dtype),
        grid=(2,),
        in_specs=[
            pl.BlockSpec(memory_space=pltpu.MemorySpace.HBM),
            pl.BlockSpec(memory_space=pltpu.MemorySpace.HBM),
            pl.BlockSpec((1, co_half), lambda i: (0, i)),
            pl.BlockSpec((1, co_half), lambda i: (0, i)),
        ],
        out_specs=pl.BlockSpec(memory_space=pltpu.MemorySpace.HBM),
        scratch_shapes=[
            pltpu.VMEM((n, kj, _LANES), jnp.float32),
            pltpu.VMEM((n, c_in), jnp.bfloat16),
            pltpu.VMEM((_N_SUB, sub, kj, _LANES), jnp.float32),
            pltpu.VMEM((2, n, sub), jnp.float32),
            pltpu.SemaphoreType.DMA,
            pltpu.SemaphoreType.DMA((_N_SUB,)),
            pltpu.SemaphoreType.DMA((2,)),
        ],
        compiler_params=pltpu.CompilerParams(
            dimension_semantics=("parallel",),
            vmem_limit_bytes=56 * 1024 * 1024,
        ),
    )(x3, w3, gamma2, beta2)


# speedup vs baseline: 1.0551x; 1.0551x over previous
import functools

import jax
import jax.numpy as jnp
from jax.experimental import pallas as pl
from jax.experimental.pallas import tpu as pltpu

_LANES = 128
_N_SUB = 8


def _bnneck_kernel(x_ref, w_ref, gamma_ref, beta_ref, o_ref, x_asm, *, n):
    c_in = x_ref.shape[1] * _LANES

    @pl.when(pl.program_id(1) == 0)
    def _cache_x():
        x_asm[...] = x_ref[...].astype(jnp.bfloat16).reshape(n, c_in)

    wk = w_ref[...].astype(jnp.bfloat16).reshape(w_ref.shape[0], c_in)
    y = jax.lax.dot_general(
        x_asm[...], wk, dimension_numbers=(((1,), (1,)), ((), ())),
        preferred_element_type=jnp.float32)
    inv_n = 1.0 / float(n)
    mean = jnp.sum(y, axis=0, keepdims=True) * inv_n
    diff = y - mean
    var = jnp.sum(diff * diff, axis=0, keepdims=True) * inv_n
    z = diff * jax.lax.rsqrt(var + 1e-5)
    z = z * gamma_ref[...] + beta_ref[...]
    o_ref[...] = jnp.where(z >= 0, z, 0.25 * z)


def kernel(x, weight, gamma, beta):
    n, c_in, h, w_sp = x.shape
    assert h == 1 and w_sp == 1
    c_out = weight.shape[0]
    assert n % 8 == 0 and c_in % _LANES == 0
    kj = c_in // _LANES
    tile_co = c_out // (2 * _N_SUB)
    assert tile_co % _LANES == 0

    x3 = x.reshape(n, kj, _LANES)
    w3 = weight.reshape(c_out, kj, _LANES)
    gamma2 = gamma.reshape(1, c_out).astype(jnp.float32)
    beta2 = beta.reshape(1, c_out).astype(jnp.float32)

    body = functools.partial(_bnneck_kernel, n=n)
    return pl.pallas_call(
        body,
        out_shape=jax.ShapeDtypeStruct((n, c_out), x.dtype),
        grid=(2, _N_SUB),
        in_specs=[
            pl.BlockSpec((n, kj, _LANES), lambda i, j: (0, 0, 0)),
            pl.BlockSpec((tile_co, kj, _LANES),
                         lambda i, j: (i * _N_SUB + j, 0, 0)),
            pl.BlockSpec((1, tile_co), lambda i, j: (0, i * _N_SUB + j)),
            pl.BlockSpec((1, tile_co), lambda i, j: (0, i * _N_SUB + j)),
        ],
        out_specs=pl.BlockSpec((n, tile_co), lambda i, j: (0, i * _N_SUB + j)),
        scratch_shapes=[pltpu.VMEM((n, c_in), jnp.bfloat16)],
        compiler_params=pltpu.CompilerParams(
            dimension_semantics=("parallel", "arbitrary"),
            vmem_limit_bytes=56 * 1024 * 1024,
        ),
    )(x3, w3, gamma2, beta2)


# speedup vs baseline: 1.5535x; 1.4724x over previous
import functools

import jax
import jax.numpy as jnp
from jax.experimental import pallas as pl
from jax.experimental.pallas import tpu as pltpu

_LANES = 128
_N_SUB = 2


def _bnneck_kernel(x_ref, w_ref, gamma_ref, beta_ref, o_ref, x_asm, *, n):
    c_in = x_ref.shape[1] * _LANES

    @pl.when(pl.program_id(1) == 0)
    def _cache_x():
        x_asm[...] = x_ref[...].astype(jnp.bfloat16).reshape(n, c_in)

    wk = w_ref[...].astype(jnp.bfloat16).reshape(w_ref.shape[0], c_in)
    y = jax.lax.dot_general(
        x_asm[...], wk, dimension_numbers=(((1,), (1,)), ((), ())),
        preferred_element_type=jnp.float32)
    inv_n = 1.0 / float(n)
    mean = jnp.sum(y, axis=0, keepdims=True) * inv_n
    diff = y - mean
    var = jnp.sum(diff * diff, axis=0, keepdims=True) * inv_n
    z = diff * jax.lax.rsqrt(var + 1e-5)
    z = z * gamma_ref[...] + beta_ref[...]
    o_ref[...] = jnp.where(z >= 0, z, 0.25 * z)


def kernel(x, weight, gamma, beta):
    n, c_in, h, w_sp = x.shape
    assert h == 1 and w_sp == 1
    c_out = weight.shape[0]
    assert n % 8 == 0 and c_in % _LANES == 0
    kj = c_in // _LANES
    tile_co = c_out // (2 * _N_SUB)
    assert tile_co % _LANES == 0

    x3 = x.reshape(n, kj, _LANES)
    w3 = weight.reshape(c_out, kj, _LANES)
    gamma2 = gamma.reshape(1, c_out).astype(jnp.float32)
    beta2 = beta.reshape(1, c_out).astype(jnp.float32)

    body = functools.partial(_bnneck_kernel, n=n)
    return pl.pallas_call(
        body,
        out_shape=jax.ShapeDtypeStruct((n, c_out), x.dtype),
        grid=(2, _N_SUB),
        in_specs=[
            pl.BlockSpec((n, kj, _LANES), lambda i, j: (0, 0, 0)),
            pl.BlockSpec((tile_co, kj, _LANES),
                         lambda i, j: (i * _N_SUB + j, 0, 0)),
            pl.BlockSpec((1, tile_co), lambda i, j: (0, i * _N_SUB + j)),
            pl.BlockSpec((1, tile_co), lambda i, j: (0, i * _N_SUB + j)),
        ],
        out_specs=pl.BlockSpec((n, tile_co), lambda i, j: (0, i * _N_SUB + j)),
        scratch_shapes=[pltpu.VMEM((n, c_in), jnp.bfloat16)],
        compiler_params=pltpu.CompilerParams(
            dimension_semantics=("parallel", "arbitrary"),
            vmem_limit_bytes=56 * 1024 * 1024,
        ),
    )(x3, w3, gamma2, beta2)
